# baseline (speedup 1.0000x reference)
import functools

import jax
import jax.numpy as jnp
from jax import lax
from jax.experimental import pallas as pl
from jax.experimental.pallas import tpu as pltpu
from jax.experimental.pallas import tpu_sc as plsc

R = 128
C = 32768


_J = 16
_B = C // _J


def _argmax_body(x_ref, out_ref, m_ref):
    j = pl.program_id(0)
    x = x_ref[...]
    m = jnp.max(x, axis=1, keepdims=True)
    ii = lax.broadcasted_iota(jnp.int32, x.shape, 1)
    li = jnp.min(jnp.where(x == m, ii, _B), axis=1, keepdims=True)

    @pl.when(j == 0)
    def _():
        m_ref[...] = m
        out_ref[...] = li

    @pl.when(j > 0)
    def _():
        better = m > m_ref[...]
        out_ref[...] = jnp.where(better, li + j * _B, out_ref[...])
        m_ref[...] = jnp.maximum(m_ref[...], m)


_argmax_tc = pl.pallas_call(
    _argmax_body,
    grid=(_J,),
    in_specs=[pl.BlockSpec((R, _B), lambda j: (0, j))],
    out_specs=pl.BlockSpec((R, 1), lambda j: (0, 0)),
    out_shape=jax.ShapeDtypeStruct((R, 1), jnp.int32),
    scratch_shapes=[pltpu.VMEM((R, 1), jnp.float32)],
)


_NC = 2
_NS = 16
_L = 16
_NW = _NC * _NS
_RPW = R // _NW

def _onehot_sc_body(idx_hbm, out_hbm, zrow, idxv, posv, onesv, zsem, ssem):
    w = lax.axis_index("c") * _NS + lax.axis_index("s")
    base_row = w * _RPW

    def zbody(i, carry):
        zrow[pl.ds(i * _L, _L)] = jnp.zeros((_L,), jnp.float32)
        return carry

    lax.fori_loop(0, C // _L, zbody, 0)

    copies = [
        pltpu.async_copy(
            zrow,
            out_hbm.at[pl.ds(pl.multiple_of((base_row + j) * C, C), C)],
            zsem,
        )
        for j in range(_RPW)
    ]
    for cp in copies:
        cp.wait()
    plsc.subcore_barrier()

    @pl.when(lax.rem(w, _L // _RPW) == 0)
    def _():
        pltpu.sync_copy(idx_hbm.at[pl.ds(pl.multiple_of(base_row, _L), _L)], idxv)
        lanes = lax.iota(jnp.int32, _L)
        posv[...] = (base_row + lanes) * C + idxv[...]
        onesv[...] = jnp.ones((_L,), jnp.float32)
        pltpu.async_copy(onesv, out_hbm.at[posv], ssem).wait()


@functools.lru_cache(maxsize=1)
def _build_onehot_sc():
    mesh = plsc.VectorSubcoreMesh(
        core_axis_name="c", subcore_axis_name="s",
        num_cores=_NC, num_subcores=_NS,
    )
    return pl.kernel(
        _onehot_sc_body,
        out_type=jax.ShapeDtypeStruct((R * C,), jnp.float32),
        mesh=mesh,
        scratch_types=[
            pltpu.VMEM((C,), jnp.float32),
            pltpu.VMEM((_L,), jnp.int32),
            pltpu.VMEM((_L,), jnp.int32),
            pltpu.VMEM((_L,), jnp.float32),
            pltpu.SemaphoreType.DMA,
            pltpu.SemaphoreType.DMA,
        ],
    )


def kernel(vec):
    idx = _argmax_tc(vec).reshape(R)
    return _build_onehot_sc()(idx).reshape(R, C)


# speedup vs baseline: 1.1492x; 1.1492x over previous
import functools

import jax
import jax.numpy as jnp
from jax import lax
from jax.experimental import pallas as pl
from jax.experimental.pallas import tpu as pltpu
from jax.experimental.pallas import tpu_sc as plsc

R = 128
C = 32768


_J = 16
_B = C // _J


def _argmax_body(x_ref, out_ref, m_ref):
    j = pl.program_id(0)
    x = x_ref[...]
    m = jnp.max(x, axis=1, keepdims=True)
    ii = lax.broadcasted_iota(jnp.int32, x.shape, 1)
    li = jnp.min(jnp.where(x == m, ii, _B), axis=1, keepdims=True)

    @pl.when(j == 0)
    def _():
        m_ref[...] = m
        out_ref[...] = li

    @pl.when(j > 0)
    def _():
        better = m > m_ref[...]
        out_ref[...] = jnp.where(better, li + j * _B, out_ref[...])
        m_ref[...] = jnp.maximum(m_ref[...], m)


_argmax_tc = pl.pallas_call(
    _argmax_body,
    grid=(_J,),
    in_specs=[pl.BlockSpec((R, _B), lambda j: (0, j))],
    out_specs=pl.BlockSpec((R, 1), lambda j: (0, 0)),
    out_shape=jax.ShapeDtypeStruct((R, 1), jnp.int32),
    scratch_shapes=[pltpu.VMEM((R, 1), jnp.float32)],
)


_NC = 2
_NS = 16
_L = 16
_NW = _NC * _NS
_RPW = R // _NW
_ZBUF = 4096

def _onehot_sc_body(idx_hbm, out_hbm, zrow, idxv, posv, onesv, zsem, ssem):
    w = lax.axis_index("c") * _NS + lax.axis_index("s")
    base_row = w * _RPW

    zk = _ZBUF // (_L * 16)

    def zbody(i, carry):
        for k in range(16):
            zrow[pl.ds((i * 16 + k) * _L, _L)] = jnp.zeros((_L,), jnp.float32)
        return carry

    lax.fori_loop(0, zk, zbody, 0, unroll=True)

    copies = [
        pltpu.async_copy(
            zrow,
            out_hbm.at[pl.ds(pl.multiple_of(base_row * C + j * _ZBUF, _ZBUF), _ZBUF)],
            zsem,
        )
        for j in range(_RPW * C // _ZBUF)
    ]
    for cp in copies:
        cp.wait()
    plsc.subcore_barrier()

    @pl.when(lax.rem(w, _L // _RPW) == 0)
    def _():
        pltpu.sync_copy(idx_hbm.at[pl.ds(pl.multiple_of(base_row, _L), _L)], idxv)
        lanes = lax.iota(jnp.int32, _L)
        posv[...] = (base_row + lanes) * C + idxv[...]
        onesv[...] = jnp.ones((_L,), jnp.float32)
        pltpu.async_copy(onesv, out_hbm.at[posv], ssem).wait()


@functools.lru_cache(maxsize=1)
def _build_onehot_sc():
    mesh = plsc.VectorSubcoreMesh(
        core_axis_name="c", subcore_axis_name="s",
        num_cores=_NC, num_subcores=_NS,
    )
    return pl.kernel(
        _onehot_sc_body,
        out_type=jax.ShapeDtypeStruct((R * C,), jnp.float32),
        mesh=mesh,
        scratch_types=[
            pltpu.VMEM((_ZBUF,), jnp.float32),
            pltpu.VMEM((_L,), jnp.int32),
            pltpu.VMEM((_L,), jnp.int32),
            pltpu.VMEM((_L,), jnp.float32),
            pltpu.SemaphoreType.DMA,
            pltpu.SemaphoreType.DMA,
        ],
    )


def kernel(vec):
    idx = _argmax_tc(vec).reshape(R)
    return _build_onehot_sc()(idx).reshape(R, C)


# speedup vs baseline: 1.6957x; 1.4756x over previous
import functools

import jax
import jax.numpy as jnp
from jax import lax
from jax.experimental import pallas as pl
from jax.experimental.pallas import tpu as pltpu
from jax.experimental.pallas import tpu_sc as plsc

R = 128
C = 32768


_J = 16
_B = C // _J


def _argmax_body(x_ref, out_ref, m_ref):
    j = pl.program_id(0)
    x = x_ref[...]
    m = jnp.max(x, axis=1, keepdims=True)
    ii = lax.broadcasted_iota(jnp.int32, x.shape, 1)
    li = jnp.min(jnp.where(x == m, ii, _B), axis=1, keepdims=True)

    @pl.when(j == 0)
    def _():
        m_ref[...] = m
        out_ref[...] = li

    @pl.when(j > 0)
    def _():
        better = m > m_ref[...]
        out_ref[...] = jnp.where(better, li + j * _B, out_ref[...])
        m_ref[...] = jnp.maximum(m_ref[...], m)


_argmax_tc = pl.pallas_call(
    _argmax_body,
    grid=(_J,),
    in_specs=[pl.BlockSpec((R, _B), lambda j: (0, j))],
    out_specs=pl.BlockSpec((R, 1), lambda j: (0, 0)),
    out_shape=jax.ShapeDtypeStruct((R, 1), jnp.int32),
    scratch_shapes=[pltpu.VMEM((R, 1), jnp.float32)],
)


_NC = 2
_NS = 16
_L = 16
_NW = _NC * _NS
_RPW = R // _NW
_ZW = 2048
_SEG = 128


def _onehot_sc_body(idx_hbm, out_hbm, zbuf, idxv, ohbuf, zsem, osem):
    w = lax.axis_index("c") * _NS + lax.axis_index("s")
    base_row = pl.multiple_of(w * _RPW, _RPW)

    def zbody(i, carry):
        for k in range(4):
            for q in range(_RPW):
                zbuf[q, pl.ds((i * 4 + k) * _L, _L)] = jnp.zeros(
                    (_L,), jnp.float32
                )
        return carry

    lax.fori_loop(0, _ZW // (4 * _L), zbody, 0, unroll=True)

    copies = [
        pltpu.async_copy(
            zbuf,
            out_hbm.at[pl.ds(base_row, _RPW), pl.ds(k * _ZW, _ZW)],
            zsem,
        )
        for k in range(C // _ZW)
    ]
    for cp in copies:
        cp.wait()

    chunk_base = pl.multiple_of((w // (_L // _RPW)) * _L, _L)
    pltpu.sync_copy(idx_hbm.at[pl.ds(chunk_base, _L)], idxv.at[pl.ds(0, _L)])
    lanes = lax.iota(jnp.int32, _L)
    ocopies = []
    for j in range(_RPW):
        lane = (base_row - chunk_base) + j
        i = idxv[pl.ds(lane, _L)][0]
        seg = pl.multiple_of((i // _SEG) * _SEG, _SEG)
        within = i - seg
        for t in range(_SEG // _L):
            oh = jnp.where(lanes + t * _L == within, 1.0, 0.0)
            ohbuf[j, pl.ds(t * _L, _L)] = oh.astype(jnp.float32)
        ocopies.append(
            pltpu.async_copy(
                ohbuf.at[pl.ds(j, 1)],
                out_hbm.at[pl.ds(base_row + j, 1), pl.ds(seg, _SEG)],
                osem,
            )
        )
    for cp in ocopies:
        cp.wait()


@functools.lru_cache(maxsize=1)
def _build_onehot_sc():
    mesh = plsc.VectorSubcoreMesh(
        core_axis_name="c", subcore_axis_name="s",
        num_cores=_NC, num_subcores=_NS,
    )
    return pl.kernel(
        _onehot_sc_body,
        out_type=jax.ShapeDtypeStruct((R, C), jnp.float32),
        mesh=mesh,
        scratch_types=[
            pltpu.VMEM((_RPW, _ZW), jnp.float32),
            pltpu.VMEM((2 * _L,), jnp.int32),
            pltpu.VMEM((_RPW, _SEG), jnp.float32),
            pltpu.SemaphoreType.DMA,
            pltpu.SemaphoreType.DMA,
        ],
    )


def kernel(vec):
    idx = _argmax_tc(vec).reshape(R)
    return _build_onehot_sc()(idx)


# speedup vs baseline: 1.9401x; 1.1441x over previous
import functools

import jax
import jax.numpy as jnp
from jax import lax
from jax.experimental import pallas as pl
from jax.experimental.pallas import tpu as pltpu
from jax.experimental.pallas import tpu_sc as plsc

R = 128
C = 32768


_J = 8
_B = C // _J


def _argmax_body(x_ref, out_ref, m_ref):
    j = pl.program_id(0)
    x = x_ref[...]
    m = jnp.max(x, axis=1, keepdims=True)
    ii = lax.broadcasted_iota(jnp.int32, x.shape, 1)
    li = jnp.min(jnp.where(x == m, ii, _B), axis=1, keepdims=True)

    @pl.when(j == 0)
    def _():
        m_ref[...] = m
        out_ref[...] = li

    @pl.when(j > 0)
    def _():
        better = m > m_ref[...]
        out_ref[...] = jnp.where(better, li + j * _B, out_ref[...])
        m_ref[...] = jnp.maximum(m_ref[...], m)


_argmax_tc = pl.pallas_call(
    _argmax_body,
    grid=(_J,),
    in_specs=[pl.BlockSpec((R, _B), lambda j: (0, j))],
    out_specs=pl.BlockSpec((R, 1), lambda j: (0, 0)),
    out_shape=jax.ShapeDtypeStruct((R, 1), jnp.int32),
    scratch_shapes=[pltpu.VMEM((R, 1), jnp.float32)],
)


_NC = 2
_NS = 16
_L = 16
_NW = _NC * _NS
_RPW = R // _NW
_ZW = 2048
_SEG = 128


def _onehot_sc_body(idx_hbm, out_hbm, zbuf, idxv, ohbuf, zsem, osem):
    w = lax.axis_index("c") * _NS + lax.axis_index("s")
    base_row = pl.multiple_of(w * _RPW, _RPW)

    def zbody(i, carry):
        for k in range(4):
            for q in range(_RPW):
                zbuf[q, pl.ds((i * 4 + k) * _L, _L)] = jnp.zeros(
                    (_L,), jnp.float32
                )
        return carry

    lax.fori_loop(0, _ZW // (4 * _L), zbody, 0, unroll=True)

    copies = [
        pltpu.async_copy(
            zbuf,
            out_hbm.at[pl.ds(base_row, _RPW), pl.ds(k * _ZW, _ZW)],
            zsem,
        )
        for k in range(C // _ZW)
    ]
    for cp in copies:
        cp.wait()

    chunk_base = pl.multiple_of((w // (_L // _RPW)) * _L, _L)
    pltpu.sync_copy(idx_hbm.at[pl.ds(chunk_base, _L)], idxv.at[pl.ds(0, _L)])
    lanes = lax.iota(jnp.int32, _L)
    ocopies = []
    for j in range(_RPW):
        lane = (base_row - chunk_base) + j
        i = idxv[pl.ds(lane, _L)][0]
        seg = pl.multiple_of((i // _SEG) * _SEG, _SEG)
        within = i - seg
        for t in range(_SEG // _L):
            oh = jnp.where(lanes + t * _L == within, 1.0, 0.0)
            ohbuf[j, pl.ds(t * _L, _L)] = oh.astype(jnp.float32)
        ocopies.append(
            pltpu.async_copy(
                ohbuf.at[pl.ds(j, 1)],
                out_hbm.at[pl.ds(base_row + j, 1), pl.ds(seg, _SEG)],
                osem,
            )
        )
    for cp in ocopies:
        cp.wait()


@functools.lru_cache(maxsize=1)
def _build_onehot_sc():
    mesh = plsc.VectorSubcoreMesh(
        core_axis_name="c", subcore_axis_name="s",
        num_cores=_NC, num_subcores=_NS,
    )
    return pl.kernel(
        _onehot_sc_body,
        out_type=jax.ShapeDtypeStruct((R, C), jnp.float32),
        mesh=mesh,
        scratch_types=[
            pltpu.VMEM((_RPW, _ZW), jnp.float32),
            pltpu.VMEM((2 * _L,), jnp.int32),
            pltpu.VMEM((_RPW, _SEG), jnp.float32),
            pltpu.SemaphoreType.DMA,
            pltpu.SemaphoreType.DMA,
        ],
    )


def kernel(vec):
    idx = _argmax_tc(vec).reshape(R)
    return _build_onehot_sc()(idx)


# speedup vs baseline: 2.0395x; 1.0512x over previous
import functools

import jax
import jax.numpy as jnp
from jax import lax
from jax.experimental import pallas as pl
from jax.experimental.pallas import tpu as pltpu
from jax.experimental.pallas import tpu_sc as plsc

R = 128
C = 32768


_J = 4
_B = C // _J


def _argmax_body(x_ref, out_ref, m_ref):
    j = pl.program_id(0)
    x = x_ref[...]
    m = jnp.max(x, axis=1, keepdims=True)
    ii = lax.broadcasted_iota(jnp.int32, x.shape, 1)
    li = jnp.min(jnp.where(x == m, ii, _B), axis=1, keepdims=True)

    @pl.when(j == 0)
    def _():
        m_ref[...] = m
        out_ref[...] = li

    @pl.when(j > 0)
    def _():
        better = m > m_ref[...]
        out_ref[...] = jnp.where(better, li + j * _B, out_ref[...])
        m_ref[...] = jnp.maximum(m_ref[...], m)


_argmax_tc = pl.pallas_call(
    _argmax_body,
    grid=(_J,),
    in_specs=[pl.BlockSpec((R, _B), lambda j: (0, j))],
    out_specs=pl.BlockSpec((R, 1), lambda j: (0, 0)),
    out_shape=jax.ShapeDtypeStruct((R, 1), jnp.int32),
    scratch_shapes=[pltpu.VMEM((R, 1), jnp.float32)],
)


_NC = 2
_NS = 16
_L = 16
_NW = _NC * _NS
_RPW = R // _NW
_ZW = 2048
_SEG = 128


def _onehot_sc_body(idx_hbm, out_hbm, zbuf, idxv, ohbuf, zsem, osem):
    w = lax.axis_index("c") * _NS + lax.axis_index("s")
    base_row = pl.multiple_of(w * _RPW, _RPW)

    def zbody(i, carry):
        for k in range(4):
            for q in range(_RPW):
                zbuf[q, pl.ds((i * 4 + k) * _L, _L)] = jnp.zeros(
                    (_L,), jnp.float32
                )
        return carry

    lax.fori_loop(0, _ZW // (4 * _L), zbody, 0, unroll=True)

    copies = [
        pltpu.async_copy(
            zbuf,
            out_hbm.at[pl.ds(base_row, _RPW), pl.ds(k * _ZW, _ZW)],
            zsem,
        )
        for k in range(C // _ZW)
    ]
    for cp in copies:
        cp.wait()

    chunk_base = pl.multiple_of((w // (_L // _RPW)) * _L, _L)
    pltpu.sync_copy(idx_hbm.at[pl.ds(chunk_base, _L)], idxv.at[pl.ds(0, _L)])
    lanes = lax.iota(jnp.int32, _L)
    ocopies = []
    for j in range(_RPW):
        lane = (base_row - chunk_base) + j
        i = idxv[pl.ds(lane, _L)][0]
        seg = pl.multiple_of((i // _SEG) * _SEG, _SEG)
        within = i - seg
        for t in range(_SEG // _L):
            oh = jnp.where(lanes + t * _L == within, 1.0, 0.0)
            ohbuf[j, pl.ds(t * _L, _L)] = oh.astype(jnp.float32)
        ocopies.append(
            pltpu.async_copy(
                ohbuf.at[pl.ds(j, 1)],
                out_hbm.at[pl.ds(base_row + j, 1), pl.ds(seg, _SEG)],
                osem,
            )
        )
    for cp in ocopies:
        cp.wait()


@functools.lru_cache(maxsize=1)
def _build_onehot_sc():
    mesh = plsc.VectorSubcoreMesh(
        core_axis_name="c", subcore_axis_name="s",
        num_cores=_NC, num_subcores=_NS,
    )
    return pl.kernel(
        _onehot_sc_body,
        out_type=jax.ShapeDtypeStruct((R, C), jnp.float32),
        mesh=mesh,
        scratch_types=[
            pltpu.VMEM((_RPW, _ZW), jnp.float32),
            pltpu.VMEM((2 * _L,), jnp.int32),
            pltpu.VMEM((_RPW, _SEG), jnp.float32),
            pltpu.SemaphoreType.DMA,
            pltpu.SemaphoreType.DMA,
        ],
    )


def kernel(vec):
    idx = _argmax_tc(vec).reshape(R)
    return _build_onehot_sc()(idx)


# speedup vs baseline: 2.0435x; 1.0020x over previous
import functools

import jax
import jax.numpy as jnp
from jax import lax
from jax.experimental import pallas as pl
from jax.experimental.pallas import tpu as pltpu
from jax.experimental.pallas import tpu_sc as plsc

R = 128
C = 32768


_J = 2
_B = C // _J


def _argmax_body(x_ref, out_ref, m_ref):
    j = pl.program_id(0)
    x = x_ref[...]
    m = jnp.max(x, axis=1, keepdims=True)
    ii = lax.broadcasted_iota(jnp.int32, x.shape, 1)
    li = jnp.min(jnp.where(x == m, ii, _B), axis=1, keepdims=True)

    @pl.when(j == 0)
    def _():
        m_ref[...] = m
        out_ref[...] = li

    @pl.when(j > 0)
    def _():
        better = m > m_ref[...]
        out_ref[...] = jnp.where(better, li + j * _B, out_ref[...])
        m_ref[...] = jnp.maximum(m_ref[...], m)


_argmax_tc = pl.pallas_call(
    _argmax_body,
    grid=(_J,),
    in_specs=[pl.BlockSpec((R, _B), lambda j: (0, j))],
    out_specs=pl.BlockSpec((R, 1), lambda j: (0, 0)),
    out_shape=jax.ShapeDtypeStruct((R, 1), jnp.int32),
    scratch_shapes=[pltpu.VMEM((R, 1), jnp.float32)],
)


_NC = 2
_NS = 16
_L = 16
_NW = _NC * _NS
_RPW = R // _NW
_ZW = 2048
_SEG = 128


def _onehot_sc_body(idx_hbm, out_hbm, zbuf, idxv, ohbuf, zsem, osem):
    w = lax.axis_index("c") * _NS + lax.axis_index("s")
    base_row = pl.multiple_of(w * _RPW, _RPW)

    def zbody(i, carry):
        for k in range(4):
            for q in range(_RPW):
                zbuf[q, pl.ds((i * 4 + k) * _L, _L)] = jnp.zeros(
                    (_L,), jnp.float32
                )
        return carry

    lax.fori_loop(0, _ZW // (4 * _L), zbody, 0, unroll=True)

    copies = [
        pltpu.async_copy(
            zbuf,
            out_hbm.at[pl.ds(base_row, _RPW), pl.ds(k * _ZW, _ZW)],
            zsem,
        )
        for k in range(C // _ZW)
    ]
    for cp in copies:
        cp.wait()

    chunk_base = pl.multiple_of((w // (_L // _RPW)) * _L, _L)
    pltpu.sync_copy(idx_hbm.at[pl.ds(chunk_base, _L)], idxv.at[pl.ds(0, _L)])
    lanes = lax.iota(jnp.int32, _L)
    ocopies = []
    for j in range(_RPW):
        lane = (base_row - chunk_base) + j
        i = idxv[pl.ds(lane, _L)][0]
        seg = pl.multiple_of((i // _SEG) * _SEG, _SEG)
        within = i - seg
        for t in range(_SEG // _L):
            oh = jnp.where(lanes + t * _L == within, 1.0, 0.0)
            ohbuf[j, pl.ds(t * _L, _L)] = oh.astype(jnp.float32)
        ocopies.append(
            pltpu.async_copy(
                ohbuf.at[pl.ds(j, 1)],
                out_hbm.at[pl.ds(base_row + j, 1), pl.ds(seg, _SEG)],
                osem,
            )
        )
    for cp in ocopies:
        cp.wait()


@functools.lru_cache(maxsize=1)
def _build_onehot_sc():
    mesh = plsc.VectorSubcoreMesh(
        core_axis_name="c", subcore_axis_name="s",
        num_cores=_NC, num_subcores=_NS,
    )
    return pl.kernel(
        _onehot_sc_body,
        out_type=jax.ShapeDtypeStruct((R, C), jnp.float32),
        mesh=mesh,
        scratch_types=[
            pltpu.VMEM((_RPW, _ZW), jnp.float32),
            pltpu.VMEM((2 * _L,), jnp.int32),
            pltpu.VMEM((_RPW, _SEG), jnp.float32),
            pltpu.SemaphoreType.DMA,
            pltpu.SemaphoreType.DMA,
        ],
    )


def kernel(vec):
    idx = _argmax_tc(vec).reshape(R)
    return _build_onehot_sc()(idx)


# speedup vs baseline: 2.0725x; 1.0142x over previous
import functools

import jax
import jax.numpy as jnp
from jax import lax
from jax.experimental import pallas as pl
from jax.experimental.pallas import tpu as pltpu
from jax.experimental.pallas import tpu_sc as plsc

R = 128
C = 32768


_J = 2
_B = C // _J


def _argmax_body(x_ref, out_ref, m_ref):
    j = pl.program_id(0)
    x = x_ref[...]
    m = jnp.max(x, axis=1, keepdims=True)
    ii = lax.broadcasted_iota(jnp.int32, x.shape, 1)
    li = jnp.min(jnp.where(x == m, ii, _B), axis=1, keepdims=True)

    @pl.when(j == 0)
    def _():
        m_ref[...] = m
        out_ref[...] = li

    @pl.when(j > 0)
    def _():
        better = m > m_ref[...]
        out_ref[...] = jnp.where(better, li + j * _B, out_ref[...])
        m_ref[...] = jnp.maximum(m_ref[...], m)


_argmax_tc = pl.pallas_call(
    _argmax_body,
    grid=(_J,),
    in_specs=[pl.BlockSpec((R, _B), lambda j: (0, j))],
    out_specs=pl.BlockSpec((R, 1), lambda j: (0, 0)),
    out_shape=jax.ShapeDtypeStruct((R, 1), jnp.int32),
    scratch_shapes=[pltpu.VMEM((R, 1), jnp.float32)],
)


_NC = 2
_NS = 16
_L = 16
_NW = _NC * _NS
_RPW = R // _NW
_ZW = 2048
_SEG = 128


def _onehot_sc_body(idx_hbm, out_hbm, zbuf, idxv, ohbuf, zsem, osem):
    w = lax.axis_index("c") * _NS + lax.axis_index("s")
    base_row = pl.multiple_of(w * _RPW, _RPW)

    def zbody(i, carry):
        for k in range(4):
            for q in range(_RPW):
                zbuf[q, pl.ds((i * 4 + k) * _L, _L)] = jnp.zeros(
                    (_L,), jnp.float32
                )
        return carry

    lax.fori_loop(0, _ZW // (4 * _L), zbody, 0, unroll=False)

    copies = [
        pltpu.async_copy(
            zbuf,
            out_hbm.at[pl.ds(base_row, _RPW), pl.ds(k * _ZW, _ZW)],
            zsem,
        )
        for k in range(C // _ZW)
    ]
    for cp in copies:
        cp.wait()

    chunk_base = pl.multiple_of((w // (_L // _RPW)) * _L, _L)
    pltpu.sync_copy(idx_hbm.at[pl.ds(chunk_base, _L)], idxv.at[pl.ds(0, _L)])
    lanes = lax.iota(jnp.int32, _L)
    ocopies = []
    for j in range(_RPW):
        lane = (base_row - chunk_base) + j
        i = idxv[pl.ds(lane, _L)][0]
        seg = pl.multiple_of((i // _SEG) * _SEG, _SEG)
        within = i - seg
        for t in range(_SEG // _L):
            oh = jnp.where(lanes + t * _L == within, 1.0, 0.0)
            ohbuf[j, pl.ds(t * _L, _L)] = oh.astype(jnp.float32)
        ocopies.append(
            pltpu.async_copy(
                ohbuf.at[pl.ds(j, 1)],
                out_hbm.at[pl.ds(base_row + j, 1), pl.ds(seg, _SEG)],
                osem,
            )
        )
    for cp in ocopies:
        cp.wait()


@functools.lru_cache(maxsize=1)
def _build_onehot_sc():
    mesh = plsc.VectorSubcoreMesh(
        core_axis_name="c", subcore_axis_name="s",
        num_cores=_NC, num_subcores=_NS,
    )
    return pl.kernel(
        _onehot_sc_body,
        out_type=jax.ShapeDtypeStruct((R, C), jnp.float32),
        mesh=mesh,
        scratch_types=[
            pltpu.VMEM((_RPW, _ZW), jnp.float32),
            pltpu.VMEM((2 * _L,), jnp.int32),
            pltpu.VMEM((_RPW, _SEG), jnp.float32),
            pltpu.SemaphoreType.DMA,
            pltpu.SemaphoreType.DMA,
        ],
    )


def kernel(vec):
    idx = _argmax_tc(vec).reshape(R)
    return _build_onehot_sc()(idx)


# speedup vs baseline: 2.2533x; 1.0872x over previous
import functools

import jax
import jax.numpy as jnp
from jax import lax
from jax.experimental import pallas as pl
from jax.experimental.pallas import tpu as pltpu
from jax.experimental.pallas import tpu_sc as plsc

R = 128
C = 32768


_J = 2
_B = C // _J


def _argmax_body(x_ref, out_ref, m_ref):
    j = pl.program_id(0)
    x = x_ref[...]
    m = jnp.max(x, axis=1, keepdims=True)
    ii = lax.broadcasted_iota(jnp.int32, x.shape, 1)
    li = jnp.min(jnp.where(x == m, ii, _B), axis=1, keepdims=True)

    @pl.when(j == 0)
    def _():
        m_ref[...] = m
        out_ref[...] = li

    @pl.when(j > 0)
    def _():
        better = m > m_ref[...]
        out_ref[...] = jnp.where(better, li + j * _B, out_ref[...])
        m_ref[...] = jnp.maximum(m_ref[...], m)


_argmax_tc = pl.pallas_call(
    _argmax_body,
    grid=(_J,),
    in_specs=[pl.BlockSpec((R, _B), lambda j: (0, j))],
    out_specs=pl.BlockSpec((R, 1), lambda j: (0, 0)),
    out_shape=jax.ShapeDtypeStruct((R, 1), jnp.int32),
    scratch_shapes=[pltpu.VMEM((R, 1), jnp.float32)],
)


_NC = 2
_NS = 16
_L = 16
_NW = _NC * _NS
_RPW = R // _NW
_ZW = 2048
_SEG = 128


def _zero_sc_body(out_hbm, zbuf, zsem):
    w = lax.axis_index("c") * _NS + lax.axis_index("s")
    base_row = pl.multiple_of(w * _RPW, _RPW)

    def zbody(i, carry):
        for k in range(4):
            for q in range(_RPW):
                zbuf[q, pl.ds((i * 4 + k) * _L, _L)] = jnp.zeros(
                    (_L,), jnp.float32
                )
        return carry

    lax.fori_loop(0, _ZW // (4 * _L), zbody, 0, unroll=False)

    def fire(k, carry):
        pltpu.async_copy(
            zbuf,
            out_hbm.at[
                pl.ds(base_row, _RPW),
                pl.ds(pl.multiple_of(k * _ZW, _ZW), _ZW),
            ],
            zsem,
        )
        return carry

    lax.fori_loop(0, C // _ZW, fire, 0, unroll=False)

    def drain(k, carry):
        pltpu.make_async_copy(
            zbuf,
            out_hbm.at[pl.ds(base_row, _RPW), pl.ds(0, _ZW)],
            zsem,
        ).wait()
        return carry

    lax.fori_loop(0, C // _ZW, drain, 0, unroll=False)


@functools.lru_cache(maxsize=1)
def _build_zero_sc():
    mesh = plsc.VectorSubcoreMesh(
        core_axis_name="c", subcore_axis_name="s",
        num_cores=_NC, num_subcores=_NS,
    )
    return pl.kernel(
        _zero_sc_body,
        out_type=jax.ShapeDtypeStruct((R, C), jnp.float32),
        mesh=mesh,
        scratch_types=[
            pltpu.VMEM((_RPW, _ZW), jnp.float32),
            pltpu.SemaphoreType.DMA,
        ],
    )


def _patch_body(idx_smem, idx_vmem, z_ref, out_ref, pbuf, sem):
    del z_ref
    iv = idx_vmem[...]
    within = lax.rem(iv, _SEG)
    cols = lax.broadcasted_iota(jnp.int32, (R, _SEG), 1)
    pbuf[...] = (cols == within).astype(jnp.float32)
    for r in range(R):
        seg = pl.multiple_of((idx_smem[r, 0] // _SEG) * _SEG, _SEG)
        pltpu.async_copy(
            pbuf.at[pl.ds(r, 1)],
            out_ref.at[pl.ds(r, 1), pl.ds(seg, _SEG)],
            sem,
        )
    pltpu.make_async_copy(pbuf, out_ref.at[pl.ds(0, R), pl.ds(0, _SEG)], sem).wait()


_patch_tc = pl.pallas_call(
    _patch_body,
    in_specs=[
        pl.BlockSpec(memory_space=pltpu.SMEM),
        pl.BlockSpec(memory_space=pltpu.VMEM),
        pl.BlockSpec(memory_space=pl.ANY),
    ],
    out_specs=pl.BlockSpec(memory_space=pl.ANY),
    out_shape=jax.ShapeDtypeStruct((R, C), jnp.float32),
    input_output_aliases={2: 0},
    scratch_shapes=[
        pltpu.VMEM((R, _SEG), jnp.float32),
        pltpu.SemaphoreType.DMA,
    ],
)


def kernel(vec):
    zeros = _build_zero_sc()()
    idx = _argmax_tc(vec)
    return _patch_tc(idx, idx, zeros)
